# baseline (speedup 1.0000x reference)
import functools

import jax
import jax.numpy as jnp
from jax import lax
from jax.experimental import pallas as pl
from jax.experimental.pallas import tpu as pltpu
from jax.experimental.pallas import tpu_sc as plsc

D = 64
SCALE = 8.0

NC, NS, L = 2, 16, 16
NW = NC * NS
B_TOTAL = 4096 * 200
B_PER_W = B_TOTAL // NW
G_ROWS = 128
CHUNK = 640
G_PER_CHUNK = CHUNK // G_ROWS
N_CHUNK = B_PER_W // CHUNK

_mesh = plsc.VectorSubcoreMesh(core_axis_name="c", subcore_axis_name="s")


@functools.partial(
    pl.kernel,
    out_type=jax.ShapeDtypeStruct((B_TOTAL, D), jnp.float32),
    mesh=_mesh,
    scratch_types=[
        pltpu.VMEM((B_PER_W,), jnp.int32),
        pltpu.VMEM((2, CHUNK, D), jnp.float32),
        pltpu.SemaphoreType.DMA,
        pltpu.SemaphoreType.DMA,
    ],
    compiler_params=pltpu.CompilerParams(use_tc_tiling_on_sc=False),
)
def _emb_lookup(idx_hbm, table_hbm, out_hbm, idx_v, rows_v, sem0, sem1):
    wid = lax.axis_index("s") * NC + lax.axis_index("c")
    base = wid * B_PER_W

    pltpu.sync_copy(idx_hbm.at[pl.ds(base, B_PER_W)], idx_v)

    sems = (sem0, sem1)

    def issue_chunk(g, buf, sem):
        for j in range(G_PER_CHUNK):
            off = g * CHUNK + j * G_ROWS
            pltpu.async_copy(
                table_hbm.at[idx_v.at[pl.ds(off, G_ROWS)]],
                buf.at[pl.ds(j * G_ROWS, G_ROWS)],
                sem,
            )

    def wait_chunk(buf, sem):
        pltpu.make_async_copy(table_hbm.at[pl.ds(0, CHUNK)], buf, sem).wait()

    def scale_chunk(buf):
        def body(i, carry):
            for j in range(D // L):
                sl = (i, pl.ds(j * L, L))
                buf[sl] = buf[sl] * SCALE
            return carry

        lax.fori_loop(0, CHUNK, body, 0, unroll=2)

    issue_chunk(0, rows_v.at[0], sems[0])

    def super_body(s, carry):
        g0 = s * 2
        for b in range(2):
            g = g0 + b
            buf = rows_v.at[b]
            nxt = rows_v.at[1 - b]

            @pl.when(g + 1 < N_CHUNK)
            def _():
                issue_chunk(g + 1, nxt, sems[1 - b])

            wait_chunk(buf, sems[b])
            scale_chunk(buf)
            pltpu.sync_copy(buf, out_hbm.at[pl.ds(base + g * CHUNK, CHUNK)])
        return carry

    lax.fori_loop(0, N_CHUNK // 2, super_body, 0)


def kernel(x, table):
    idx = x.reshape(-1).astype(jnp.int32)
    out = _emb_lookup(idx, table)
    return out.reshape(x.shape + (D,))
